# Initial kernel scaffold; baseline (speedup 1.0000x reference)
import jax
import jax.numpy as jnp
from jax.experimental import pallas as pl


def kernel(logits, u):
    raise NotImplementedError("write your pallas kernel here")


# speedup vs baseline: 112.2907x; 112.2907x over previous
import jax
import jax.numpy as jnp
from jax.experimental import pallas as pl

_T = 0.1
_K = 8
_B = 128
_N = 32768
_ROWS_PER_BLOCK = 8


def _body(logits_ref, u_ref, d_ref, c_ref):
    tiny = jnp.finfo(jnp.float32).tiny
    lg = logits_ref[...]
    u = u_ref[...]
    z = -jnp.log(-jnp.log(jnp.clip(u, tiny, 1.0 - tiny)))
    w = lg + z
    csum = jnp.zeros_like(w)
    onehot = jnp.zeros_like(w)
    for _ in range(_K):
        w = w + jnp.log(jnp.clip(1.0 - onehot, tiny, 1.0 - tiny))
        t = w / _T
        m = jnp.max(t, axis=-1, keepdims=True)
        e = jnp.exp(t - m)
        s = jnp.sum(e, axis=-1, keepdims=True)
        onehot = e / s
        csum = csum + onehot
    c_ref[...] = csum

    neginf = jnp.float32(-jnp.inf)
    rem = lg
    rows = lg.shape[0]
    total = jnp.zeros((rows, 1), jnp.int32)
    thr = jnp.full((rows, 1), neginf, jnp.float32)
    for _ in range(_K):
        m = jnp.max(rem, axis=-1, keepdims=True)
        eq = rem == m
        cnt = jnp.sum(eq.astype(jnp.int32), axis=-1, keepdims=True)
        done = total >= _K
        thr = jnp.where(done, thr, m)
        total = total + jnp.where(done, 0, cnt)
        rem = jnp.where(eq, neginf, rem)
    d_ref[...] = (lg >= thr).astype(jnp.float32)


def kernel(logits, u):
    lg = logits.reshape(_B, _N)
    uu = u.reshape(_B, _N)
    grid = (_B // _ROWS_PER_BLOCK,)
    spec = pl.BlockSpec((_ROWS_PER_BLOCK, _N), lambda i: (i, 0))
    d, c = pl.pallas_call(
        _body,
        grid=grid,
        in_specs=[spec, spec],
        out_specs=[spec, spec],
        out_shape=[jax.ShapeDtypeStruct((_B, _N), jnp.float32)] * 2,
    )(lg, uu)
    return (d, c)
